# baseline (speedup 1.0000x reference)
import functools
import jax
import jax.numpy as jnp
from jax import lax
from jax.experimental import pallas as pl
from jax.experimental.pallas import tpu as pltpu
from jax.experimental.pallas import tpu_sc as plsc

N = 8192
M = 8192
NC = 2
NS = 16
NW = NC * NS
RP = N // NW
L = 16
NCHUNK = M // L


def _sc_body(xg_hbm, xq_hbm, yt_hbm, yq_hbm, rowpart_hbm, colpart_hbm,
             x0_v, x1_v, x2_v, xn_v, y0_v, y1_v, y2_v, yn_v, cmin_v,
             rpart_v):
    wid = lax.axis_index("s") * NC + lax.axis_index("c")

    xb = wid * 3 * RP
    pltpu.sync_copy(xg_hbm.at[pl.ds(xb, RP)], x0_v)
    pltpu.sync_copy(xg_hbm.at[pl.ds(xb + RP, RP)], x1_v)
    pltpu.sync_copy(xg_hbm.at[pl.ds(xb + 2 * RP, RP)], x2_v)
    pltpu.sync_copy(yt_hbm.at[pl.ds(0, M)], y0_v)
    pltpu.sync_copy(yt_hbm.at[pl.ds(M, M)], y1_v)
    pltpu.sync_copy(yt_hbm.at[pl.ds(2 * M, M)], y2_v)

    def prep_chunk(c, _):
        sl = pl.ds(c * L, L)
        y0 = y0_v[sl]
        y1 = y1_v[sl]
        y2 = y2_v[sl]
        yn_v[sl] = (y0 * y0 + y1 * y1) + y2 * y2
        cmin_v[sl] = jnp.full((L,), jnp.inf, jnp.float32)
        return _
    lax.fori_loop(0, NCHUNK, prep_chunk, 0)

    def prep_x(c, _):
        sl = pl.ds(c * L, L)
        x0 = x0_v[sl]
        x1 = x1_v[sl]
        x2 = x2_v[sl]
        xn_v[sl] = (x0 * x0 + x1 * x1) + x2 * x2
        return _
    lax.fori_loop(0, RP // L, prep_x, 0)

    pltpu.sync_copy(xq_hbm.at[pl.ds(xb, RP)], x0_v)
    pltpu.sync_copy(xq_hbm.at[pl.ds(xb + RP, RP)], x1_v)
    pltpu.sync_copy(xq_hbm.at[pl.ds(xb + 2 * RP, RP)], x2_v)
    pltpu.sync_copy(yq_hbm.at[pl.ds(0, M)], y0_v)
    pltpu.sync_copy(yq_hbm.at[pl.ds(M, M)], y1_v)
    pltpu.sync_copy(yq_hbm.at[pl.ds(2 * M, M)], y2_v)

    def init_rpart(i, _):
        rpart_v[pl.ds(i * L, L)] = jnp.full((L,), jnp.inf, jnp.float32)
        return _
    lax.fori_loop(0, RP, init_rpart, 0)

    def chunk_body(c, _):
        sl = pl.ds(c * L, L)
        y0 = y0_v[sl]
        y1 = y1_v[sl]
        y2 = y2_v[sl]
        yn = yn_v[sl]
        cmin0 = cmin_v[sl]

        def pred_body(p, cmin):
            base = p * L
            xnc = xn_v[pl.ds(base, L)]
            x0c = x0_v[pl.ds(base, L)]
            x1c = x1_v[pl.ds(base, L)]
            x2c = x2_v[pl.ds(base, L)]
            for u in range(L):
                s = yn + xnc[u]
                t = (x0c[u] * y0 + x1c[u] * y1) + x2c[u] * y2
                d = s + t
                cmin = jnp.minimum(cmin, d)
                rsl = pl.ds((base + u) * L, L)
                rpart_v[rsl] = jnp.minimum(rpart_v[rsl], d)
            return cmin

        cmin = lax.fori_loop(0, RP // L, pred_body, cmin0)
        cmin_v[sl] = cmin
        return _
    lax.fori_loop(0, NCHUNK, chunk_body, 0)

    pltpu.sync_copy(rpart_v, rowpart_hbm.at[pl.ds(wid * RP * L, RP * L)])
    pltpu.sync_copy(cmin_v, colpart_hbm.at[pl.ds(wid * M, M)])


def _sc_call(xg, xq, yt, yq):
    mesh = plsc.VectorSubcoreMesh(core_axis_name="c", subcore_axis_name="s")
    f = functools.partial(
        pl.kernel,
        mesh=mesh,
        out_type=[
            jax.ShapeDtypeStruct((NW * RP * L,), jnp.float32),
            jax.ShapeDtypeStruct((NW * M,), jnp.float32),
        ],
        scratch_types=[
            pltpu.VMEM((RP,), jnp.float32),
            pltpu.VMEM((RP,), jnp.float32),
            pltpu.VMEM((RP,), jnp.float32),
            pltpu.VMEM((RP,), jnp.float32),
            pltpu.VMEM((M,), jnp.float32),
            pltpu.VMEM((M,), jnp.float32),
            pltpu.VMEM((M,), jnp.float32),
            pltpu.VMEM((M,), jnp.float32),
            pltpu.VMEM((M,), jnp.float32),
            pltpu.VMEM((RP * L,), jnp.float32),
        ],
    )(_sc_body)
    return f(xg, xq, yt, yq)


def _combine_kernel(rowpart_ref, colpart_ref, out_ref):
    rowmin = jnp.min(rowpart_ref[...], axis=1)
    colmin = jnp.min(colpart_ref[...], axis=0)
    mr = jnp.sum(rowmin) / N
    mc = jnp.sum(colmin) / M
    out_ref[...] = jnp.full((1, 1), (mr + mc) * 0.5, jnp.float32)


def kernel(pred_positions, target_positions):
    def arrange_x(a):
        return a.T.reshape(3, NW, RP).transpose(1, 0, 2).reshape(NW * 3 * RP)
    xg = arrange_x(pred_positions)
    xq = arrange_x(lax.reduce_precision(pred_positions, 8, 7) * -2.0)
    yt = target_positions.T.reshape(3 * M)
    yq = lax.reduce_precision(target_positions, 8, 7).T.reshape(3 * M)
    rowpart, colpart = _sc_call(xg, xq, yt, yq)
    out = pl.pallas_call(
        _combine_kernel,
        out_shape=jax.ShapeDtypeStruct((1, 1), jnp.float32),
    )(rowpart.reshape(N, L), colpart.reshape(NW, M))
    return out[0, 0]


# speedup vs baseline: 5.6131x; 5.6131x over previous
import functools
import jax
import jax.numpy as jnp
from jax import lax
from jax.experimental import pallas as pl
from jax.experimental.pallas import tpu as pltpu
from jax.experimental.pallas import tpu_sc as plsc

N = 8192
M = 8192
KSC = 512
MTC = M - KSC
BJ = 1280
NJ = MTC // BJ

NC = 2
NS = 16
NW = NC * NS
RP = N // NW
L = 16
NCHUNK = KSC // L


def _tc_kernel(x_ref, yt_ref, rowmin_ref, colsum_ref, xm_ref, xn_ref,
               colacc_ref):
    j = pl.program_id(0)
    nj = pl.num_programs(0)

    @pl.when(j == 0)
    def _init():
        x = x_ref[...]
        xm_ref[...] = x * -2.0
        xn_ref[...] = jnp.sum(x * x, axis=1, keepdims=True)
        rowmin_ref[...] = jnp.full_like(rowmin_ref, jnp.inf)
        colacc_ref[0, 0] = 0.0

    yt = yt_ref[...]
    yn = jnp.sum(yt * yt, axis=0, keepdims=True)
    d = (xn_ref[...] + yn) + jnp.dot(xm_ref[...], yt)

    rowmin_ref[...] = jnp.minimum(rowmin_ref[...], jnp.min(d, axis=1, keepdims=True))
    colacc_ref[0, 0] += jnp.sum(jnp.min(d, axis=0))

    @pl.when(j == nj - 1)
    def _finish():
        colsum_ref[...] = jnp.full((1, 1), colacc_ref[0, 0], jnp.float32)


def _tc_call(x, yt_tc):
    return pl.pallas_call(
        _tc_kernel,
        grid=(NJ,),
        in_specs=[
            pl.BlockSpec((N, 3), lambda j: (0, 0)),
            pl.BlockSpec((3, BJ), lambda j: (0, j)),
        ],
        out_specs=[
            pl.BlockSpec((N, 1), lambda j: (0, 0)),
            pl.BlockSpec((1, 1), lambda j: (0, 0)),
        ],
        out_shape=[
            jax.ShapeDtypeStruct((N, 1), jnp.float32),
            jax.ShapeDtypeStruct((1, 1), jnp.float32),
        ],
        scratch_shapes=[
            pltpu.VMEM((N, 3), jnp.float32),
            pltpu.VMEM((N, 1), jnp.float32),
            pltpu.SMEM((1, 1), jnp.float32),
        ],
    )(x, yt_tc)


def _sc_body(xg_hbm, xq_hbm, yt_hbm, yq_hbm, rowpart_hbm, colpart_hbm,
             x0_v, x1_v, x2_v, xn_v, y0_v, y1_v, y2_v, yn_v, cmin_v,
             rpart_v):
    wid = lax.axis_index("s") * NC + lax.axis_index("c")

    xb = wid * 3 * RP
    pltpu.sync_copy(xg_hbm.at[pl.ds(xb, RP)], x0_v)
    pltpu.sync_copy(xg_hbm.at[pl.ds(xb + RP, RP)], x1_v)
    pltpu.sync_copy(xg_hbm.at[pl.ds(xb + 2 * RP, RP)], x2_v)
    pltpu.sync_copy(yt_hbm.at[pl.ds(0, KSC)], y0_v)
    pltpu.sync_copy(yt_hbm.at[pl.ds(KSC, KSC)], y1_v)
    pltpu.sync_copy(yt_hbm.at[pl.ds(2 * KSC, KSC)], y2_v)

    def prep_chunk(c, _):
        sl = pl.ds(c * L, L)
        y0 = y0_v[sl]
        y1 = y1_v[sl]
        y2 = y2_v[sl]
        yn_v[sl] = (y0 * y0 + y1 * y1) + y2 * y2
        cmin_v[sl] = jnp.full((L,), jnp.inf, jnp.float32)
        return _
    lax.fori_loop(0, NCHUNK, prep_chunk, 0)

    def prep_x(c, _):
        sl = pl.ds(c * L, L)
        x0 = x0_v[sl]
        x1 = x1_v[sl]
        x2 = x2_v[sl]
        xn_v[sl] = (x0 * x0 + x1 * x1) + x2 * x2
        return _
    lax.fori_loop(0, RP // L, prep_x, 0)

    pltpu.sync_copy(xq_hbm.at[pl.ds(xb, RP)], x0_v)
    pltpu.sync_copy(xq_hbm.at[pl.ds(xb + RP, RP)], x1_v)
    pltpu.sync_copy(xq_hbm.at[pl.ds(xb + 2 * RP, RP)], x2_v)
    pltpu.sync_copy(yq_hbm.at[pl.ds(0, KSC)], y0_v)
    pltpu.sync_copy(yq_hbm.at[pl.ds(KSC, KSC)], y1_v)
    pltpu.sync_copy(yq_hbm.at[pl.ds(2 * KSC, KSC)], y2_v)

    def init_rpart(i, _):
        rpart_v[pl.ds(i * L, L)] = jnp.full((L,), jnp.inf, jnp.float32)
        return _
    lax.fori_loop(0, RP, init_rpart, 0)

    def chunk_body(c, _):
        sl = pl.ds(c * L, L)
        y0 = y0_v[sl]
        y1 = y1_v[sl]
        y2 = y2_v[sl]
        yn = yn_v[sl]
        cmin0 = cmin_v[sl]

        def pred_body(p, cmin):
            base = p * L
            xnc = xn_v[pl.ds(base, L)]
            x0c = x0_v[pl.ds(base, L)]
            x1c = x1_v[pl.ds(base, L)]
            x2c = x2_v[pl.ds(base, L)]
            for u in range(L):
                s = yn + xnc[u]
                t = (x0c[u] * y0 + x1c[u] * y1) + x2c[u] * y2
                d = s + t
                cmin = jnp.minimum(cmin, d)
                rsl = pl.ds((base + u) * L, L)
                rpart_v[rsl] = jnp.minimum(rpart_v[rsl], d)
            return cmin

        cmin = lax.fori_loop(0, RP // L, pred_body, cmin0)
        cmin_v[sl] = cmin
        return _
    lax.fori_loop(0, NCHUNK, chunk_body, 0)

    pltpu.sync_copy(rpart_v, rowpart_hbm.at[pl.ds(wid * RP * L, RP * L)])
    pltpu.sync_copy(cmin_v, colpart_hbm.at[pl.ds(wid * KSC, KSC)])


def _sc_call(xg, xq, yt, yq):
    mesh = plsc.VectorSubcoreMesh(core_axis_name="c", subcore_axis_name="s")
    f = functools.partial(
        pl.kernel,
        mesh=mesh,
        out_type=[
            jax.ShapeDtypeStruct((NW * RP * L,), jnp.float32),
            jax.ShapeDtypeStruct((NW * KSC,), jnp.float32),
        ],
        scratch_types=[
            pltpu.VMEM((RP,), jnp.float32),
            pltpu.VMEM((RP,), jnp.float32),
            pltpu.VMEM((RP,), jnp.float32),
            pltpu.VMEM((RP,), jnp.float32),
            pltpu.VMEM((KSC,), jnp.float32),
            pltpu.VMEM((KSC,), jnp.float32),
            pltpu.VMEM((KSC,), jnp.float32),
            pltpu.VMEM((KSC,), jnp.float32),
            pltpu.VMEM((KSC,), jnp.float32),
            pltpu.VMEM((RP * L,), jnp.float32),
        ],
    )(_sc_body)
    return f(xg, xq, yt, yq)


def _combine_kernel(rowmin_tc_ref, colsum_tc_ref, rowpart_ref, colpart_ref,
                    out_ref):
    rowmin_sc = jnp.min(rowpart_ref[...], axis=1, keepdims=True)
    rowmin = jnp.minimum(rowmin_tc_ref[...], rowmin_sc)
    colmin_sc = jnp.min(colpart_ref[...], axis=0)
    mr = jnp.sum(rowmin) / N
    mc = (colsum_tc_ref[0, 0] + jnp.sum(colmin_sc)) / M
    out_ref[...] = jnp.full((1, 1), (mr + mc) * 0.5, jnp.float32)


def kernel(pred_positions, target_positions):
    y_sc = target_positions[MTC:]

    def arrange_x(a):
        return a.T.reshape(3, NW, RP).transpose(1, 0, 2).reshape(NW * 3 * RP)
    xg = arrange_x(pred_positions)
    xq = arrange_x(lax.reduce_precision(pred_positions, 8, 7) * -2.0)
    yt_sc = y_sc.T.reshape(3 * KSC)
    yq_sc = lax.reduce_precision(y_sc, 8, 7).T.reshape(3 * KSC)

    rowpart, colpart = _sc_call(xg, xq, yt_sc, yq_sc)

    yt_tc = target_positions.T[:, :MTC]
    rowmin_tc, colsum_tc = _tc_call(pred_positions, yt_tc)

    out = pl.pallas_call(
        _combine_kernel,
        out_shape=jax.ShapeDtypeStruct((1, 1), jnp.float32),
    )(rowmin_tc, colsum_tc, rowpart.reshape(N, L), colpart.reshape(NW, KSC))
    return out[0, 0]


# speedup vs baseline: 8.1761x; 1.4566x over previous
import jax
import jax.numpy as jnp
from jax import lax
from jax.experimental import pallas as pl
from jax.experimental.pallas import tpu as pltpu

N = 8192
M = 8192
BJ = 1024


def _chamfer_kernel(x_ref, y_ref, out_ref, xm_ref, xn_ref, yts_ref,
                    rowmin_ref, colacc_ref):
    j = pl.program_id(0)
    nj = pl.num_programs(0)

    @pl.when(j == 0)
    def _init():
        x = x_ref[...]
        xm_ref[...] = x * -2.0
        xn_ref[...] = jnp.sum(x * x, axis=1, keepdims=True)
        yts_ref[...] = lax.transpose(y_ref[...], (1, 0))
        rowmin_ref[...] = jnp.full_like(rowmin_ref, jnp.inf)
        colacc_ref[0, 0] = 0.0

    yt = yts_ref[:, pl.ds(j * BJ, BJ)]
    yn = jnp.sum(yt * yt, axis=0, keepdims=True)
    d = (xn_ref[...] + yn) + jnp.dot(xm_ref[...], yt)

    rowmin_ref[...] = jnp.minimum(rowmin_ref[...], jnp.min(d, axis=1, keepdims=True))
    colacc_ref[0, 0] += jnp.sum(jnp.min(d, axis=0))

    @pl.when(j == nj - 1)
    def _finish():
        mean_row = jnp.sum(rowmin_ref[...]) / N
        mean_col = colacc_ref[0, 0] / M
        out_ref[...] = jnp.full((1, 1), (mean_row + mean_col) * 0.5, jnp.float32)


def kernel(pred_positions, target_positions):
    out = pl.pallas_call(
        _chamfer_kernel,
        grid=(M // BJ,),
        in_specs=[
            pl.BlockSpec((N, 3), lambda j: (0, 0)),
            pl.BlockSpec((M, 3), lambda j: (0, 0)),
        ],
        out_specs=pl.BlockSpec((1, 1), lambda j: (0, 0)),
        out_shape=jax.ShapeDtypeStruct((1, 1), jnp.float32),
        scratch_shapes=[
            pltpu.VMEM((N, 3), jnp.float32),
            pltpu.VMEM((N, 1), jnp.float32),
            pltpu.VMEM((3, M), jnp.float32),
            pltpu.VMEM((N, 1), jnp.float32),
            pltpu.SMEM((1, 1), jnp.float32),
        ],
    )(pred_positions, target_positions)
    return out[0, 0]
